# baseline (speedup 1.0000x reference)
import jax
import jax.numpy as jnp
from jax import lax
from jax.experimental import pallas as pl
from jax.experimental.pallas import tpu as pltpu
from jax.experimental.pallas import tpu_sc as plsc

E = 320000
N = 10000
N_PAD = 10240
D = 128

NC = 2
NS = 16
NW = NC * NS
EPW = E // NW
CHUNK = 128
NFULL = EPW // CHUNK
REM = EPW - NFULL * CHUNK
NBLK = N_PAD // CHUNK
BPT = NBLK // NS
NPT = N_PAD // NS


def _sc_body(from_hbm, idx_hbm, zacc_hbm, z640_hbm, ones_hbm,
             sums_hbm, cnts_hbm,
             acc, cnt, rows2, idx2, rows16, idx16,
             ones_v, ones16_v, c640_v, isem, rsem, ssem, csem):
    cid = lax.axis_index("c")
    sid = lax.axis_index("s")
    wid = sid * NC + cid
    ebase = wid * EPW

    pltpu.sync_copy(z640_hbm, c640_v)
    pltpu.sync_copy(ones_hbm, ones_v)
    pltpu.sync_copy(ones_hbm.at[pl.ds(0, REM)], ones16_v)

    def zstep(t, carry):
        r0 = pl.multiple_of((sid + t * NS) * CHUNK, CHUNK)
        pltpu.sync_copy(zacc_hbm.at[pl.ds(r0, CHUNK)], acc.at[pl.ds(r0, CHUNK)])
        return carry

    lax.fori_loop(0, BPT, zstep, 0)
    pltpu.sync_copy(c640_v, cnt.at[pl.ds(sid * NPT, NPT)])
    plsc.subcore_barrier()

    pltpu.async_copy(idx_hbm.at[pl.ds(ebase, CHUNK)], idx2.at[0], isem.at[0])
    pltpu.async_copy(from_hbm.at[pl.ds(ebase, CHUNK)], rows2.at[0], rsem.at[0])
    pltpu.async_copy(idx_hbm.at[pl.ds(ebase + CHUNK, CHUNK)], idx2.at[1],
                     isem.at[1])
    pltpu.async_copy(from_hbm.at[pl.ds(ebase + CHUNK, CHUNK)], rows2.at[1],
                     rsem.at[1])
    pltpu.make_async_copy(idx_hbm.at[pl.ds(ebase, CHUNK)], idx2.at[0],
                          isem.at[0]).wait()
    pltpu.make_async_copy(from_hbm.at[pl.ds(ebase, CHUNK)], rows2.at[0],
                          rsem.at[0]).wait()
    pltpu.async_copy(rows2.at[0], acc.at[idx2.at[0]], ssem.at[0], add=True)
    pltpu.async_copy(ones_v, cnt.at[idx2.at[0]], csem.at[0], add=True)

    def step(j, carry):
        s = lax.rem(j, 2)
        ns = lax.rem(j + 1, 2)
        base = ebase + j * CHUNK
        pltpu.make_async_copy(rows2.at[ns], acc.at[idx2.at[ns]],
                              ssem.at[ns]).wait()
        pltpu.make_async_copy(ones_v, cnt.at[idx2.at[ns]],
                              csem.at[ns]).wait()
        pltpu.async_copy(idx_hbm.at[pl.ds(base + CHUNK, CHUNK)], idx2.at[ns],
                         isem.at[ns])
        pltpu.async_copy(from_hbm.at[pl.ds(base + CHUNK, CHUNK)],
                         rows2.at[ns], rsem.at[ns])
        pltpu.make_async_copy(idx_hbm.at[pl.ds(base, CHUNK)], idx2.at[s],
                              isem.at[s]).wait()
        pltpu.make_async_copy(from_hbm.at[pl.ds(base, CHUNK)], rows2.at[s],
                              rsem.at[s]).wait()
        pltpu.async_copy(rows2.at[s], acc.at[idx2.at[s]], ssem.at[s],
                         add=True)
        pltpu.async_copy(ones_v, cnt.at[idx2.at[s]], csem.at[s], add=True)
        return carry

    lax.fori_loop(1, NFULL - 1, step, 0)

    ps = (NFULL - 2) % 2
    ls = (NFULL - 1) % 2
    pltpu.make_async_copy(rows2.at[ps], acc.at[idx2.at[ps]],
                          ssem.at[ps]).wait()
    pltpu.make_async_copy(ones_v, cnt.at[idx2.at[ps]], csem.at[ps]).wait()
    lbase = ebase + (NFULL - 1) * CHUNK
    pltpu.make_async_copy(idx_hbm.at[pl.ds(lbase, CHUNK)], idx2.at[ls],
                          isem.at[ls]).wait()
    pltpu.make_async_copy(from_hbm.at[pl.ds(lbase, CHUNK)], rows2.at[ls],
                          rsem.at[ls]).wait()
    pltpu.sync_copy(rows2.at[ls], acc.at[idx2.at[ls]], add=True)
    pltpu.sync_copy(ones_v, cnt.at[idx2.at[ls]], add=True)

    rbase = ebase + NFULL * CHUNK
    pltpu.sync_copy(idx_hbm.at[pl.ds(rbase, REM)], idx16.at[0])
    pltpu.sync_copy(from_hbm.at[pl.ds(rbase, REM)], rows16)
    pltpu.sync_copy(rows16, acc.at[idx16.at[0]], add=True)
    pltpu.sync_copy(ones16_v, cnt.at[idx16.at[0]], add=True)
    plsc.subcore_barrier()

    def dstep(t, carry):
        r0 = pl.multiple_of((sid + t * NS) * CHUNK, CHUNK)
        pltpu.sync_copy(acc.at[pl.ds(r0, CHUNK)],
                        sums_hbm.at[cid, pl.ds(r0, CHUNK)])
        return carry

    lax.fori_loop(0, BPT, dstep, 0)
    pltpu.sync_copy(cnt.at[pl.ds(sid * NPT, NPT)], c640_v)
    pltpu.sync_copy(c640_v, cnts_hbm.at[pl.ds(cid * N_PAD + sid * NPT, NPT)])


_sc_scatter = pl.kernel(
    _sc_body,
    out_type=(
        jax.ShapeDtypeStruct((NC, N_PAD, D), jnp.float32),
        jax.ShapeDtypeStruct((NC * N_PAD,), jnp.float32),
    ),
    mesh=plsc.VectorSubcoreMesh(
        core_axis_name="c", subcore_axis_name="s",
        num_cores=NC, num_subcores=NS,
    ),
    scratch_types=[
        pltpu.VMEM_SHARED((N_PAD, D), jnp.float32),
        pltpu.VMEM_SHARED((N_PAD,), jnp.float32),
        pltpu.VMEM((2, CHUNK, D), jnp.float32),
        pltpu.VMEM((2, CHUNK), jnp.int32),
        pltpu.VMEM((REM, D), jnp.float32),
        pltpu.VMEM((1, REM), jnp.int32),
        pltpu.VMEM((CHUNK,), jnp.float32),
        pltpu.VMEM((REM,), jnp.float32),
        pltpu.VMEM((NPT,), jnp.float32),
        pltpu.SemaphoreType.DMA((2,)),
        pltpu.SemaphoreType.DMA((2,)),
        pltpu.SemaphoreType.DMA((2,)),
        pltpu.SemaphoreType.DMA((2,)),
    ],
)


def _tc_body(s_ref, c_ref, w_ref, b_ref, o_ref):
    s = s_ref[0] + s_ref[1]
    cr = c_ref[0, 0] + c_ref[1, 0]
    row = lax.broadcasted_iota(jnp.int32, (128, 128), 0)
    col = lax.broadcasted_iota(jnp.int32, (128, 128), 1)
    eye = jnp.where(row == col, 1.0, 0.0).astype(jnp.float32)
    ct = lax.dot_general(eye, cr, (((1,), (1,)), ((), ())),
                         preferred_element_type=jnp.float32)
    mean = s / jnp.maximum(ct, 1.0)
    y = lax.dot_general(mean, w_ref[...], (((1,), (1,)), ((), ())),
                        preferred_element_type=jnp.float32)
    o_ref[...] = jnp.maximum(y + b_ref[...], 0.0)


def _tc_finish(sums, cnts, W, b2):
    bn = 128
    cnts3 = cnts.reshape(NC, N_PAD // 128, 1, 128)
    return pl.pallas_call(
        _tc_body,
        grid=(N_PAD // bn,),
        in_specs=[
            pl.BlockSpec((NC, bn, D), lambda i: (0, i, 0)),
            pl.BlockSpec((NC, 1, 1, 128), lambda i: (0, i, 0, 0)),
            pl.BlockSpec((D, D), lambda i: (0, 0)),
            pl.BlockSpec((1, D), lambda i: (0, 0)),
        ],
        out_specs=pl.BlockSpec((bn, D), lambda i: (i, 0)),
        out_shape=jax.ShapeDtypeStruct((N_PAD, D), jnp.float32),
    )(sums, cnts3, W, b2)


@jax.jit
def _run(from_tensor, to_index, W, b):
    idx = to_index.astype(jnp.int32)
    zacc = jnp.zeros((N_PAD, D), jnp.float32)
    z640 = jnp.zeros((NPT,), jnp.float32)
    ones = jnp.ones((CHUNK,), jnp.float32)
    sums, cnts = _sc_scatter(from_tensor, idx, zacc, z640, ones)
    return _tc_finish(sums, cnts, W, b.reshape(1, D))[:N]


def kernel(from_tensor, to_index, dim_size, W, b):
    del dim_size
    return _run(from_tensor, to_index, W, b)


# speedup vs baseline: 1.0007x; 1.0007x over previous
import jax
import jax.numpy as jnp
from jax import lax
from jax.experimental import pallas as pl
from jax.experimental.pallas import tpu as pltpu
from jax.experimental.pallas import tpu_sc as plsc

E = 320000
N = 10000
N_PAD = 10240
D = 128

NC = 2
NS = 16
NW = NC * NS
EPW = E // NW
CHUNK = 128
NFULL = EPW // CHUNK
REM = EPW - NFULL * CHUNK
NBLK = N_PAD // CHUNK
BPT = NBLK // NS
NPT = N_PAD // NS


def _sc_body(from_hbm, idxm_hbm, idxr_hbm, zacc_hbm, z640_hbm, ones_hbm,
             sums_hbm, cnts_hbm,
             acc, cnt, rows2, idx_all, rows16, idx16,
             ones_v, ones16_v, c640_v, rsem, ssem, csem):
    cid = lax.axis_index("c")
    sid = lax.axis_index("s")
    wid = sid * NC + cid
    ebase = wid * EPW

    pltpu.sync_copy(z640_hbm, c640_v)
    pltpu.sync_copy(ones_hbm, ones_v)
    pltpu.sync_copy(ones_hbm.at[pl.ds(0, REM)], ones16_v)
    pltpu.sync_copy(idxm_hbm.at[wid], idx_all)
    pltpu.sync_copy(idxr_hbm.at[wid], idx16)

    def zstep(t, carry):
        r0 = pl.multiple_of((sid + t * NS) * CHUNK, CHUNK)
        pltpu.sync_copy(zacc_hbm.at[pl.ds(r0, CHUNK)], acc.at[pl.ds(r0, CHUNK)])
        return carry

    lax.fori_loop(0, BPT, zstep, 0)
    pltpu.sync_copy(c640_v, cnt.at[pl.ds(sid * NPT, NPT)])
    plsc.subcore_barrier()

    pltpu.async_copy(from_hbm.at[pl.ds(ebase, CHUNK)], rows2.at[0], rsem.at[0])
    pltpu.async_copy(from_hbm.at[pl.ds(ebase + CHUNK, CHUNK)], rows2.at[1],
                     rsem.at[1])
    pltpu.make_async_copy(from_hbm.at[pl.ds(ebase, CHUNK)], rows2.at[0],
                          rsem.at[0]).wait()
    pltpu.async_copy(rows2.at[0], acc.at[idx_all.at[0]], ssem.at[0], add=True)
    pltpu.async_copy(ones_v, cnt.at[idx_all.at[0]], csem, add=True)

    def step(j, carry):
        s = lax.rem(j, 2)
        ns = lax.rem(j + 1, 2)
        base = ebase + j * CHUNK
        pltpu.make_async_copy(rows2.at[ns], acc.at[idx_all.at[0]],
                              ssem.at[ns]).wait()
        pltpu.async_copy(from_hbm.at[pl.ds(base + CHUNK, CHUNK)],
                         rows2.at[ns], rsem.at[ns])
        pltpu.make_async_copy(from_hbm.at[pl.ds(base, CHUNK)], rows2.at[s],
                              rsem.at[s]).wait()
        pltpu.async_copy(rows2.at[s], acc.at[idx_all.at[j]], ssem.at[s],
                         add=True)
        pltpu.async_copy(ones_v, cnt.at[idx_all.at[j]], csem, add=True)
        return carry

    lax.fori_loop(1, NFULL - 1, step, 0)

    ps = (NFULL - 2) % 2
    ls = (NFULL - 1) % 2
    pltpu.make_async_copy(rows2.at[ps], acc.at[idx_all.at[0]],
                          ssem.at[ps]).wait()
    lbase = ebase + (NFULL - 1) * CHUNK
    pltpu.make_async_copy(from_hbm.at[pl.ds(lbase, CHUNK)], rows2.at[ls],
                          rsem.at[ls]).wait()
    pltpu.sync_copy(rows2.at[ls], acc.at[idx_all.at[NFULL - 1]], add=True)
    pltpu.async_copy(ones_v, cnt.at[idx_all.at[NFULL - 1]], csem, add=True)

    rbase = ebase + NFULL * CHUNK
    pltpu.sync_copy(from_hbm.at[pl.ds(rbase, REM)], rows16)
    pltpu.sync_copy(rows16, acc.at[idx16.at[0]], add=True)
    pltpu.sync_copy(ones16_v, cnt.at[idx16.at[0]], add=True)

    def cdrain(j, carry):
        pltpu.make_async_copy(ones_v, cnt.at[idx_all.at[0]], csem).wait()
        return carry

    lax.fori_loop(0, NFULL, cdrain, 0)
    plsc.subcore_barrier()

    def dstep(t, carry):
        r0 = pl.multiple_of((sid + t * NS) * CHUNK, CHUNK)
        pltpu.sync_copy(acc.at[pl.ds(r0, CHUNK)],
                        sums_hbm.at[cid, pl.ds(r0, CHUNK)])
        return carry

    lax.fori_loop(0, BPT, dstep, 0)
    pltpu.sync_copy(cnt.at[pl.ds(sid * NPT, NPT)], c640_v)
    pltpu.sync_copy(c640_v, cnts_hbm.at[pl.ds(cid * N_PAD + sid * NPT, NPT)])


_sc_scatter = pl.kernel(
    _sc_body,
    out_type=(
        jax.ShapeDtypeStruct((NC, N_PAD, D), jnp.float32),
        jax.ShapeDtypeStruct((NC * N_PAD,), jnp.float32),
    ),
    mesh=plsc.VectorSubcoreMesh(
        core_axis_name="c", subcore_axis_name="s",
        num_cores=NC, num_subcores=NS,
    ),
    scratch_types=[
        pltpu.VMEM_SHARED((N_PAD, D), jnp.float32),
        pltpu.VMEM_SHARED((N_PAD,), jnp.float32),
        pltpu.VMEM((2, CHUNK, D), jnp.float32),
        pltpu.VMEM((NFULL, CHUNK), jnp.int32),
        pltpu.VMEM((REM, D), jnp.float32),
        pltpu.VMEM((1, REM), jnp.int32),
        pltpu.VMEM((CHUNK,), jnp.float32),
        pltpu.VMEM((REM,), jnp.float32),
        pltpu.VMEM((NPT,), jnp.float32),
        pltpu.SemaphoreType.DMA((2,)),
        pltpu.SemaphoreType.DMA((2,)),
        pltpu.SemaphoreType.DMA,
    ],
)


def _tc_body(s_ref, c_ref, w_ref, b_ref, o_ref):
    s = s_ref[0] + s_ref[1]
    cr = c_ref[0, 0] + c_ref[1, 0]
    row = lax.broadcasted_iota(jnp.int32, (128, 128), 0)
    col = lax.broadcasted_iota(jnp.int32, (128, 128), 1)
    eye = jnp.where(row == col, 1.0, 0.0).astype(jnp.float32)
    ct = lax.dot_general(eye, cr, (((1,), (1,)), ((), ())),
                         preferred_element_type=jnp.float32)
    mean = s / jnp.maximum(ct, 1.0)
    y = lax.dot_general(mean, w_ref[...], (((1,), (1,)), ((), ())),
                        preferred_element_type=jnp.float32)
    o_ref[...] = jnp.maximum(y + b_ref[...], 0.0)


def _tc_finish(sums, cnts, W, b2):
    bn = 128
    cnts3 = cnts.reshape(NC, N_PAD // 128, 1, 128)
    return pl.pallas_call(
        _tc_body,
        grid=(pl.cdiv(N, bn),),
        in_specs=[
            pl.BlockSpec((NC, bn, D), lambda i: (0, i, 0)),
            pl.BlockSpec((NC, 1, 1, 128), lambda i: (0, i, 0, 0)),
            pl.BlockSpec((D, D), lambda i: (0, 0)),
            pl.BlockSpec((1, D), lambda i: (0, 0)),
        ],
        out_specs=pl.BlockSpec((bn, D), lambda i: (i, 0)),
        out_shape=jax.ShapeDtypeStruct((N, D), jnp.float32),
    )(sums, cnts3, W, b2)


@jax.jit
def _run(from_tensor, to_index, W, b):
    idx = to_index.astype(jnp.int32)
    idx2d = idx.reshape(NW, EPW)
    idxm = idx2d[:, :NFULL * CHUNK].reshape(NW, NFULL, CHUNK)
    idxr = idx2d[:, NFULL * CHUNK:].reshape(NW, 1, REM)
    zacc = jnp.zeros((N_PAD, D), jnp.float32)
    z640 = jnp.zeros((NPT,), jnp.float32)
    ones = jnp.ones((CHUNK,), jnp.float32)
    sums, cnts = _sc_scatter(from_tensor, idxm, idxr, zacc, z640, ones)
    return _tc_finish(sums, cnts, W, b.reshape(1, D))


def kernel(from_tensor, to_index, dim_size, W, b):
    del dim_size
    return _run(from_tensor, to_index, W, b)


# speedup vs baseline: 1.2599x; 1.2590x over previous
import jax
import jax.numpy as jnp
from jax import lax
from jax.experimental import pallas as pl
from jax.experimental.pallas import tpu as pltpu
from jax.experimental.pallas import tpu_sc as plsc

E = 320000
N = 10000
N_PAD = 10240
D = 128

NC = 2
NS = 16
NW = NC * NS
EPW = E // NW
CHUNK = 128
NFULL = EPW // CHUNK
REM = EPW - NFULL * CHUNK
NBLK = N_PAD // CHUNK
BPT = NBLK // NS
NPT = N_PAD // NS


def _sc_body(from_hbm, idxm_hbm, idxr_hbm, zacc_hbm, z640_hbm, ones_hbm,
             sums_hbm, cnts_hbm,
             acc, cnt, rows2, idx_all, rows16, idx16,
             ones_v, ones16_v, c640_v, rsem, ssem, csem):
    cid = lax.axis_index("c")
    sid = lax.axis_index("s")
    wid = sid * NC + cid
    ebase = wid * EPW

    pltpu.sync_copy(z640_hbm, c640_v)
    pltpu.sync_copy(ones_hbm, ones_v)
    pltpu.sync_copy(ones_hbm.at[pl.ds(0, REM)], ones16_v)
    pltpu.sync_copy(idxm_hbm.at[wid], idx_all)
    pltpu.sync_copy(idxr_hbm.at[wid], idx16)

    def zstep(t, carry):
        r0 = pl.multiple_of((sid + t * NS) * CHUNK, CHUNK)
        pltpu.sync_copy(zacc_hbm.at[pl.ds(r0, CHUNK)], acc.at[pl.ds(r0, CHUNK)])
        return carry

    lax.fori_loop(0, BPT, zstep, 0)
    pltpu.sync_copy(c640_v, cnt.at[pl.ds(sid * NPT, NPT)])
    plsc.subcore_barrier()

    pltpu.async_copy(from_hbm.at[pl.ds(ebase, CHUNK)], rows2.at[0], rsem.at[0])
    pltpu.async_copy(from_hbm.at[pl.ds(ebase + CHUNK, CHUNK)], rows2.at[1],
                     rsem.at[1])
    pltpu.make_async_copy(from_hbm.at[pl.ds(ebase, CHUNK)], rows2.at[0],
                          rsem.at[0]).wait()
    pltpu.async_copy(rows2.at[0], acc.at[idx_all.at[0]], ssem.at[0], add=True)
    pltpu.async_copy(ones_v, cnt.at[idx_all.at[0]], csem, add=True)

    def step(j, carry):
        s = lax.rem(j, 2)
        ns = lax.rem(j + 1, 2)
        base = ebase + j * CHUNK
        pltpu.make_async_copy(rows2.at[ns], acc.at[idx_all.at[0]],
                              ssem.at[ns]).wait()
        pltpu.async_copy(from_hbm.at[pl.ds(base + CHUNK, CHUNK)],
                         rows2.at[ns], rsem.at[ns])
        pltpu.make_async_copy(from_hbm.at[pl.ds(base, CHUNK)], rows2.at[s],
                              rsem.at[s]).wait()
        pltpu.async_copy(rows2.at[s], acc.at[idx_all.at[j]], ssem.at[s],
                         add=True)
        pltpu.async_copy(ones_v, cnt.at[idx_all.at[j]], csem, add=True)
        return carry

    lax.fori_loop(1, NFULL - 1, step, 0)

    ps = (NFULL - 2) % 2
    ls = (NFULL - 1) % 2
    pltpu.make_async_copy(rows2.at[ps], acc.at[idx_all.at[0]],
                          ssem.at[ps]).wait()
    lbase = ebase + (NFULL - 1) * CHUNK
    pltpu.make_async_copy(from_hbm.at[pl.ds(lbase, CHUNK)], rows2.at[ls],
                          rsem.at[ls]).wait()
    pltpu.sync_copy(rows2.at[ls], acc.at[idx_all.at[NFULL - 1]], add=True)
    pltpu.async_copy(ones_v, cnt.at[idx_all.at[NFULL - 1]], csem, add=True)

    rbase = ebase + NFULL * CHUNK
    pltpu.sync_copy(from_hbm.at[pl.ds(rbase, REM)], rows16)
    pltpu.sync_copy(rows16, acc.at[idx16.at[0]], add=True)
    pltpu.sync_copy(ones16_v, cnt.at[idx16.at[0]], add=True)

    def cdrain(j, carry):
        pltpu.make_async_copy(ones_v, cnt.at[idx_all.at[0]], csem).wait()
        return carry

    lax.fori_loop(0, NFULL, cdrain, 0)
    plsc.subcore_barrier()

    def dstep(t, carry):
        r0 = pl.multiple_of((sid + t * NS) * CHUNK, CHUNK)
        pltpu.sync_copy(acc.at[pl.ds(r0, CHUNK)],
                        sums_hbm.at[cid, pl.ds(r0, CHUNK)])
        return carry

    lax.fori_loop(0, BPT, dstep, 0)
    pltpu.sync_copy(cnt.at[pl.ds(sid * NPT, NPT)], c640_v)
    pltpu.sync_copy(c640_v, cnts_hbm.at[pl.ds(cid * N_PAD + sid * NPT, NPT)])


_sc_scatter = pl.kernel(
    _sc_body,
    out_type=(
        jax.ShapeDtypeStruct((NC, N_PAD, D), jnp.float32),
        jax.ShapeDtypeStruct((NC * N_PAD,), jnp.float32),
    ),
    mesh=plsc.VectorSubcoreMesh(
        core_axis_name="c", subcore_axis_name="s",
        num_cores=NC, num_subcores=NS,
    ),
    scratch_types=[
        pltpu.VMEM_SHARED((N_PAD, D), jnp.float32),
        pltpu.VMEM_SHARED((N_PAD,), jnp.float32),
        pltpu.VMEM((2, CHUNK, D), jnp.float32),
        pltpu.VMEM((NFULL, CHUNK), jnp.int32),
        pltpu.VMEM((REM, D), jnp.float32),
        pltpu.VMEM((1, REM), jnp.int32),
        pltpu.VMEM((CHUNK,), jnp.float32),
        pltpu.VMEM((REM,), jnp.float32),
        pltpu.VMEM((NPT,), jnp.float32),
        pltpu.SemaphoreType.DMA((2,)),
        pltpu.SemaphoreType.DMA((2,)),
        pltpu.SemaphoreType.DMA,
    ],
)


BN = 1024
CPB = BN // 128


def _tc_body(s_ref, c_ref, w_ref, b_ref, o_ref):
    s = s_ref[0] + s_ref[1]
    cr = c_ref[0] + c_ref[1]
    rdiv = lax.broadcasted_iota(jnp.int32, (BN, CPB), 0) // 128
    kcol = lax.broadcasted_iota(jnp.int32, (BN, CPB), 1)
    sel = jnp.where(rdiv == kcol, 1.0, 0.0).astype(jnp.float32)
    m = lax.dot_general(sel, cr, (((1,), (0,)), ((), ())),
                        preferred_element_type=jnp.float32)
    rmod = lax.broadcasted_iota(jnp.int32, (BN, D), 0) % 128
    lane = lax.broadcasted_iota(jnp.int32, (BN, D), 1)
    ct = jnp.sum(jnp.where(lane == rmod, m, 0.0), axis=1,
                 keepdims=True)
    mean = s / jnp.maximum(ct, 1.0)
    y = lax.dot_general(mean, w_ref[...], (((1,), (1,)), ((), ())),
                        preferred_element_type=jnp.float32)
    o_ref[...] = jnp.maximum(y + b_ref[...], 0.0)


def _tc_finish(sums, cnts, W, b2):
    cnts3 = cnts.reshape(NC, N_PAD // 128, 128)
    return pl.pallas_call(
        _tc_body,
        grid=(pl.cdiv(N, BN),),
        in_specs=[
            pl.BlockSpec((NC, BN, D), lambda i: (0, i, 0)),
            pl.BlockSpec((NC, CPB, 128), lambda i: (0, i, 0)),
            pl.BlockSpec((D, D), lambda i: (0, 0)),
            pl.BlockSpec((1, D), lambda i: (0, 0)),
        ],
        out_specs=pl.BlockSpec((BN, D), lambda i: (i, 0)),
        out_shape=jax.ShapeDtypeStruct((N, D), jnp.float32),
    )(sums, cnts3, W, b2)


@jax.jit
def _run(from_tensor, to_index, W, b):
    idx = to_index.astype(jnp.int32)
    idx2d = idx.reshape(NW, EPW)
    idxm = idx2d[:, :NFULL * CHUNK].reshape(NW, NFULL, CHUNK)
    idxr = idx2d[:, NFULL * CHUNK:].reshape(NW, 1, REM)
    zacc = jnp.zeros((N_PAD, D), jnp.float32)
    z640 = jnp.zeros((NPT,), jnp.float32)
    ones = jnp.ones((CHUNK,), jnp.float32)
    sums, cnts = _sc_scatter(from_tensor, idxm, idxr, zacc, z640, ones)
    return _tc_finish(sums, cnts, W, b.reshape(1, D))


def kernel(from_tensor, to_index, dim_size, W, b):
    del dim_size
    return _run(from_tensor, to_index, W, b)


# speedup vs baseline: 1.2839x; 1.0190x over previous
import jax
import jax.numpy as jnp
from jax import lax
from jax.experimental import pallas as pl
from jax.experimental.pallas import tpu as pltpu
from jax.experimental.pallas import tpu_sc as plsc

E = 320000
N = 10000
N_PAD = 10240
D = 128

NC = 2
NS = 16
NW = NC * NS
EPW = E // NW
CHUNK = 128
NFULL = EPW // CHUNK
REM = EPW - NFULL * CHUNK
NBLK = N_PAD // CHUNK
BPT = NBLK // NS
NPT = N_PAD // NS


def _sc_body(from_hbm, idxm_hbm, idxr_hbm, zacc_hbm, z640_hbm, ones_hbm,
             sums_hbm, cnts_hbm,
             acc, cnt, rows2, idx_all, rows16, idx16,
             ones_v, ones16_v, c640_v, rsem, ssem, csem):
    cid = lax.axis_index("c")
    sid = lax.axis_index("s")
    wid = sid * NC + cid
    ebase = wid * EPW

    pltpu.sync_copy(z640_hbm, c640_v)
    pltpu.sync_copy(ones_hbm, ones_v)
    pltpu.sync_copy(ones_hbm.at[pl.ds(0, REM)], ones16_v)
    pltpu.sync_copy(idxm_hbm.at[wid], idx_all)
    pltpu.sync_copy(idxr_hbm.at[wid], idx16)

    def zstep(t, carry):
        r0 = pl.multiple_of((sid + t * NS) * CHUNK, CHUNK)
        pltpu.sync_copy(zacc_hbm.at[pl.ds(r0, CHUNK)], acc.at[pl.ds(r0, CHUNK)])
        return carry

    lax.fori_loop(0, BPT, zstep, 0)
    pltpu.sync_copy(c640_v, cnt.at[pl.ds(sid * NPT, NPT)])
    plsc.subcore_barrier()

    pltpu.async_copy(from_hbm.at[pl.ds(ebase, CHUNK)], rows2.at[0], rsem.at[0])
    pltpu.async_copy(from_hbm.at[pl.ds(ebase + CHUNK, CHUNK)], rows2.at[1],
                     rsem.at[1])
    pltpu.make_async_copy(from_hbm.at[pl.ds(ebase, CHUNK)], rows2.at[0],
                          rsem.at[0]).wait()
    pltpu.async_copy(rows2.at[0], acc.at[idx_all.at[0]], ssem.at[0], add=True)
    pltpu.async_copy(ones_v, cnt.at[idx_all.at[0]], csem, add=True)

    def step(j, carry):
        s = lax.rem(j, 2)
        ns = lax.rem(j + 1, 2)
        base = ebase + j * CHUNK
        pltpu.make_async_copy(rows2.at[ns], acc.at[idx_all.at[0]],
                              ssem.at[ns]).wait()
        pltpu.async_copy(from_hbm.at[pl.ds(base + CHUNK, CHUNK)],
                         rows2.at[ns], rsem.at[ns])
        pltpu.make_async_copy(from_hbm.at[pl.ds(base, CHUNK)], rows2.at[s],
                              rsem.at[s]).wait()
        pltpu.async_copy(rows2.at[s], acc.at[idx_all.at[j]], ssem.at[s],
                         add=True)
        pltpu.async_copy(ones_v, cnt.at[idx_all.at[j]], csem, add=True)
        return carry

    lax.fori_loop(1, NFULL - 1, step, 0)

    ps = (NFULL - 2) % 2
    ls = (NFULL - 1) % 2
    pltpu.make_async_copy(rows2.at[ps], acc.at[idx_all.at[0]],
                          ssem.at[ps]).wait()
    lbase = ebase + (NFULL - 1) * CHUNK
    pltpu.make_async_copy(from_hbm.at[pl.ds(lbase, CHUNK)], rows2.at[ls],
                          rsem.at[ls]).wait()
    pltpu.sync_copy(rows2.at[ls], acc.at[idx_all.at[NFULL - 1]], add=True)
    pltpu.async_copy(ones_v, cnt.at[idx_all.at[NFULL - 1]], csem, add=True)

    rbase = ebase + NFULL * CHUNK
    pltpu.sync_copy(from_hbm.at[pl.ds(rbase, REM)], rows16)
    pltpu.sync_copy(rows16, acc.at[idx16.at[0]], add=True)
    pltpu.sync_copy(ones16_v, cnt.at[idx16.at[0]], add=True)

    def cdrain(j, carry):
        pltpu.make_async_copy(ones_v, cnt.at[idx_all.at[0]], csem).wait()
        return carry

    lax.fori_loop(0, NFULL, cdrain, 0)
    plsc.subcore_barrier()

    def dstep(t, carry):
        r0 = pl.multiple_of((sid + t * NS) * CHUNK, CHUNK)
        pltpu.sync_copy(acc.at[pl.ds(r0, CHUNK)],
                        sums_hbm.at[cid, pl.ds(r0, CHUNK)])
        return carry

    lax.fori_loop(0, BPT, dstep, 0)
    pltpu.sync_copy(cnt.at[pl.ds(sid * NPT, NPT)], c640_v)
    pltpu.sync_copy(c640_v, cnts_hbm.at[pl.ds(cid * N_PAD + sid * NPT, NPT)])


_sc_scatter = pl.kernel(
    _sc_body,
    out_type=(
        jax.ShapeDtypeStruct((NC, N_PAD, D), jnp.float32),
        jax.ShapeDtypeStruct((NC * N_PAD,), jnp.float32),
    ),
    mesh=plsc.VectorSubcoreMesh(
        core_axis_name="c", subcore_axis_name="s",
        num_cores=NC, num_subcores=NS,
    ),
    scratch_types=[
        pltpu.VMEM_SHARED((N_PAD, D), jnp.float32),
        pltpu.VMEM_SHARED((N_PAD,), jnp.float32),
        pltpu.VMEM((2, CHUNK, D), jnp.float32),
        pltpu.VMEM((NFULL, CHUNK), jnp.int32),
        pltpu.VMEM((REM, D), jnp.float32),
        pltpu.VMEM((1, REM), jnp.int32),
        pltpu.VMEM((CHUNK,), jnp.float32),
        pltpu.VMEM((REM,), jnp.float32),
        pltpu.VMEM((NPT,), jnp.float32),
        pltpu.SemaphoreType.DMA((2,)),
        pltpu.SemaphoreType.DMA((2,)),
        pltpu.SemaphoreType.DMA,
    ],
)


BN = 2048
CPB = BN // 128


def _tc_body(s_ref, c_ref, w_ref, b_ref, o_ref):
    s = s_ref[0] + s_ref[1]
    cr = c_ref[0] + c_ref[1]
    rdiv = lax.broadcasted_iota(jnp.int32, (BN, CPB), 0) // 128
    kcol = lax.broadcasted_iota(jnp.int32, (BN, CPB), 1)
    sel = jnp.where(rdiv == kcol, 1.0, 0.0).astype(jnp.float32)
    m = lax.dot_general(sel, cr, (((1,), (0,)), ((), ())),
                        preferred_element_type=jnp.float32)
    rmod = lax.broadcasted_iota(jnp.int32, (BN, D), 0) % 128
    lane = lax.broadcasted_iota(jnp.int32, (BN, D), 1)
    ct = jnp.sum(jnp.where(lane == rmod, m, 0.0), axis=1,
                 keepdims=True)
    mean = s / jnp.maximum(ct, 1.0)
    y = lax.dot_general(mean, w_ref[...], (((1,), (1,)), ((), ())),
                        preferred_element_type=jnp.float32)
    o_ref[...] = jnp.maximum(y + b_ref[...], 0.0)


def _tc_finish(sums, cnts, W, b2):
    cnts3 = cnts.reshape(NC, N_PAD // 128, 128)
    return pl.pallas_call(
        _tc_body,
        grid=(pl.cdiv(N, BN),),
        in_specs=[
            pl.BlockSpec((NC, BN, D), lambda i: (0, i, 0)),
            pl.BlockSpec((NC, CPB, 128), lambda i: (0, i, 0)),
            pl.BlockSpec((D, D), lambda i: (0, 0)),
            pl.BlockSpec((1, D), lambda i: (0, 0)),
        ],
        out_specs=pl.BlockSpec((BN, D), lambda i: (i, 0)),
        out_shape=jax.ShapeDtypeStruct((N, D), jnp.float32),
    )(sums, cnts3, W, b2)


@jax.jit
def _run(from_tensor, to_index, W, b):
    idx = to_index.astype(jnp.int32)
    idx2d = idx.reshape(NW, EPW)
    idxm = idx2d[:, :NFULL * CHUNK].reshape(NW, NFULL, CHUNK)
    idxr = idx2d[:, NFULL * CHUNK:].reshape(NW, 1, REM)
    zacc = jnp.zeros((N_PAD, D), jnp.float32)
    z640 = jnp.zeros((NPT,), jnp.float32)
    ones = jnp.ones((CHUNK,), jnp.float32)
    sums, cnts = _sc_scatter(from_tensor, idxm, idxr, zacc, z640, ones)
    return _tc_finish(sums, cnts, W, b.reshape(1, D))


def kernel(from_tensor, to_index, dim_size, W, b):
    del dim_size
    return _run(from_tensor, to_index, W, b)
